# baseline (speedup 1.0000x reference)
import functools

import jax
import jax.numpy as jnp
from jax import lax
from jax.experimental import pallas as pl
from jax.experimental.pallas import tpu as pltpu
from jax.experimental.pallas import tpu_sc as plsc

_NC = 2
_NS = 16
_NW = _NC * _NS


def _ln(z, g, b):
    m = jnp.mean(z, axis=-1, keepdims=True)
    v = jnp.var(z, axis=-1, keepdims=True)
    return (z - m) * lax.rsqrt(v + 1e-5) * g + b


def _stage_a_body(nsteps,
                  a_ref, n_ref, x_ref, sn_ref,
                  W1, b1, g1, t1, W2, b2, g2, t2,
                  Wa1, ba1, ga1, ta1, Wa2, ba2,
                  Ws1, bs1, gs1, ts1, Ws2, bs2, gs2, ts2,
                  n1_ref, ua_ref, a2_ref, nm_ref, sn2_ref):
    i = pl.program_id(0)
    a = a_ref[...]
    sn = sn_ref[...]
    sm = jnp.dot(a, sn, preferred_element_type=jnp.float32)
    n0 = n_ref[...]
    cat = jnp.concatenate([n0, sm], axis=-1)
    h = jnp.tanh(_ln(jnp.dot(cat, W1[...], preferred_element_type=jnp.float32)
                     + b1[...], g1[...], t1[...]))
    n1 = jnp.tanh(_ln(jnp.dot(h, W2[...], preferred_element_type=jnp.float32)
                      + b2[...], g2[...], t2[...])) + n0
    n1_ref[...] = n1

    cat2 = jnp.concatenate([n1, x_ref[...]], axis=-1)
    ha = jnp.tanh(_ln(jnp.dot(cat2, Wa1[...], preferred_element_type=jnp.float32)
                      + ba1[...], ga1[...], ta1[...]))
    ua = jnp.dot(ha, Wa2[...], preferred_element_type=jnp.float32) + ba2[...]
    ua_ref[...] = ua
    mx = jnp.max(ua, axis=-1, keepdims=True)
    ex = jnp.exp(ua - mx)
    a2 = ex / jnp.sum(ex, axis=-1, keepdims=True)
    a2_ref[...] = a2

    part = lax.dot_general(a2, n1, (((0,), (0,)), ((), ())),
                           preferred_element_type=jnp.float32)

    @pl.when(i == 0)
    def _():
        nm_ref[...] = part

    @pl.when(i > 0)
    def _():
        nm_ref[...] += part

    @pl.when(i == nsteps - 1)
    def _():
        nm = nm_ref[...]
        cat3 = jnp.concatenate([sn, nm], axis=-1)
        hs = jnp.tanh(_ln(jnp.dot(cat3, Ws1[...], preferred_element_type=jnp.float32)
                          + bs1[...], gs1[...], ts1[...]))
        sn2 = jnp.tanh(_ln(jnp.dot(hs, Ws2[...], preferred_element_type=jnp.float32)
                           + bs2[...], gs2[...], ts2[...])) + sn
        sn2_ref[...] = sn2


def _full(shape):
    return pl.BlockSpec(shape, lambda i: (0, 0))


def _stage_a(assignment, nodes, x, supernodes, p_node1, p_assign, p_super):
    n, s = assignment.shape
    latent = nodes.shape[1]
    xdim = x.shape[1]
    bn = 1000 if n % 1000 == 0 else n
    nsteps = n // bn
    r2 = lambda a: a.reshape(1, -1)
    pn = [p_node1[0], r2(p_node1[1]), r2(p_node1[2]), r2(p_node1[3]),
          p_node1[4], r2(p_node1[5]), r2(p_node1[6]), r2(p_node1[7])]
    pa = [p_assign[0], r2(p_assign[1]), r2(p_assign[2]), r2(p_assign[3]),
          p_assign[4], r2(p_assign[5])]
    ps = [p_super[0], r2(p_super[1]), r2(p_super[2]), r2(p_super[3]),
          p_super[4], r2(p_super[5]), r2(p_super[6]), r2(p_super[7])]
    in_specs = [
        pl.BlockSpec((bn, s), lambda i: (i, 0)),
        pl.BlockSpec((bn, latent), lambda i: (i, 0)),
        pl.BlockSpec((bn, xdim), lambda i: (i, 0)),
        _full(supernodes.shape),
    ] + [_full(w.shape) for w in pn + pa + ps]
    out_specs = [
        pl.BlockSpec((bn, latent), lambda i: (i, 0)),
        pl.BlockSpec((bn, s), lambda i: (i, 0)),
        pl.BlockSpec((bn, s), lambda i: (i, 0)),
        _full((s, latent)),
        _full((s, latent)),
    ]
    out_shape = [
        jax.ShapeDtypeStruct((n, latent), jnp.float32),
        jax.ShapeDtypeStruct((n, s), jnp.float32),
        jax.ShapeDtypeStruct((n, s), jnp.float32),
        jax.ShapeDtypeStruct((s, latent), jnp.float32),
        jax.ShapeDtypeStruct((s, latent), jnp.float32),
    ]
    fn = pl.pallas_call(
        functools.partial(_stage_a_body, nsteps),
        grid=(nsteps,),
        in_specs=in_specs,
        out_specs=out_specs,
        out_shape=out_shape,
        compiler_params=pltpu.CompilerParams(
            dimension_semantics=("arbitrary",)),
    )
    return fn(assignment, nodes, x, supernodes, *pn, *pa, *ps)


def _sc_gather(nodes1, src, dst):
    e = src.shape[0]
    latent = nodes1.shape[1]
    epw = e // _NW
    gblk = 1280
    nblk = epw // gblk
    rem = epw - nblk * gblk
    nsub = gblk // 128
    t_total = 2 * nblk

    mesh = plsc.VectorSubcoreMesh(core_axis_name="c", subcore_axis_name="s",
                                  num_cores=_NC, num_subcores=_NS)

    @functools.partial(
        pl.kernel,
        out_type=(jax.ShapeDtypeStruct((e, latent), jnp.float32),
                  jax.ShapeDtypeStruct((e, latent), jnp.float32)),
        mesh=mesh,
        scratch_types=[
            pltpu.VMEM((2, gblk), jnp.int32),
            pltpu.VMEM((gblk, latent), jnp.float32),
            pltpu.VMEM((gblk, latent), jnp.float32),
            [pltpu.SemaphoreType.DMA] * 2,
            [pltpu.SemaphoreType.DMA] * 2,
            [pltpu.SemaphoreType.DMA] * 2,
        ],
        compiler_params=pltpu.CompilerParams(use_tc_tiling_on_sc=False),
    )
    def k(tbl, srcr, dstr, hs, hd, idxv, rows0, rows1, isems, gsems, osems):
        c = lax.axis_index("c")
        s = lax.axis_index("s")
        wid = s * _NC + c
        base = wid * epw
        rows = [rows0, rows1]

        def item_off(t):
            bt = jnp.where(t < nblk, t, t - nblk)
            return base + bt * gblk

        def fire_idx(t, par):
            off = item_off(t)

            @pl.when(t < nblk)
            def _():
                pltpu.async_copy(srcr.at[pl.ds(off, gblk)],
                                 idxv.at[par], isems[par])

            @pl.when(t >= nblk)
            def _():
                pltpu.async_copy(dstr.at[pl.ds(off, gblk)],
                                 idxv.at[par], isems[par])

        def wait_idx(par):
            pltpu.make_async_copy(srcr.at[pl.ds(0, gblk)],
                                  idxv.at[par], isems[par]).wait()

        def fire_gathers(t, par):
            del t
            for j in range(nsub):
                pltpu.async_copy(
                    tbl.at[idxv.at[par, pl.ds(j * 128, 128)]],
                    rows[par].at[pl.ds(j * 128, 128), :], gsems[par])

        def wait_gathers(par):
            pltpu.make_async_copy(tbl.at[pl.ds(0, gblk)],
                                  rows[par], gsems[par]).wait()

        def fire_out(t, par):
            off = item_off(t)

            @pl.when(t < nblk)
            def _():
                pltpu.async_copy(rows[par], hs.at[pl.ds(off, gblk), :],
                                 osems[par])

            @pl.when(t >= nblk)
            def _():
                pltpu.async_copy(rows[par], hd.at[pl.ds(off, gblk), :],
                                 osems[par])

        def wait_out(par):
            pltpu.make_async_copy(tbl.at[pl.ds(0, gblk)],
                                  rows[par], osems[par]).wait()

        fire_idx(jnp.int32(0), 0)
        if t_total > 1:
            fire_idx(jnp.int32(1), 1)

        def step(t, _):
            par = lax.rem(t, 2)

            @pl.when(par == 0)
            def _():
                _step(t, 0)

            @pl.when(par == 1)
            def _():
                _step(t, 1)

            return 0

        def _step(t, par):
            wait_idx(par)

            @pl.when(t >= 2)
            def _():
                wait_out(par)

            fire_gathers(t, par)

            @pl.when(t >= 1)
            def _():
                wait_gathers(1 - par)

                @pl.when(t + 1 < t_total)
                def _():
                    fire_idx(t + 1, 1 - par)

                fire_out(t - 1, 1 - par)

        lax.fori_loop(0, t_total, step, 0)

        last = t_total - 1
        lpar = last % 2
        wait_gathers(lpar)
        fire_out(jnp.int32(last), lpar)
        wait_out(1 - lpar)
        wait_out(lpar)

        if rem:
            def do_rem(gr, out):
                off = base + nblk * gblk
                pltpu.sync_copy(gr.at[pl.ds(off, rem)],
                                idxv.at[0, pl.ds(0, rem)])
                descs = []
                p = 0
                while p < rem:
                    kk = min(128, rem - p)
                    descs.append(pltpu.async_copy(
                        tbl.at[idxv.at[0, pl.ds(p, kk)]],
                        rows0.at[pl.ds(p, kk), :], gsems[0]))
                    p += kk
                for d in descs:
                    d.wait()
                pltpu.sync_copy(rows0.at[pl.ds(0, rem), :],
                                out.at[pl.ds(off, rem), :])

            do_rem(srcr, hs)
            do_rem(dstr, hd)

    return k(nodes1, src, dst)


def _edge_body(hs_ref, hd_ref, e_ref,
               W1, b1, g1, t1, W2, b2, g2, t2, out_ref):
    latent = e_ref.shape[1]
    hs = hs_ref[...]
    hd = hd_ref[...]
    e0 = e_ref[...]
    W1f = W1[...]
    z = (jnp.dot(hs, W1f[0:latent, :], preferred_element_type=jnp.float32)
         + jnp.dot(hd, W1f[latent:2 * latent, :], preferred_element_type=jnp.float32)
         + jnp.dot(e0, W1f[2 * latent:3 * latent, :], preferred_element_type=jnp.float32)
         + b1[...])
    h = jnp.tanh(_ln(z, g1[...], t1[...]))
    out = jnp.tanh(_ln(jnp.dot(h, W2[...], preferred_element_type=jnp.float32)
                       + b2[...], g2[...], t2[...])) + e0
    out_ref[...] = out


def _edge_mlp(h_src, h_dst, edges, p_edge):
    e, latent = edges.shape
    be = 6400 if e % 6400 == 0 else e
    nsteps = e // be
    r2 = lambda a: a.reshape(1, -1)
    pe = [p_edge[0], r2(p_edge[1]), r2(p_edge[2]), r2(p_edge[3]),
          p_edge[4], r2(p_edge[5]), r2(p_edge[6]), r2(p_edge[7])]
    in_specs = [pl.BlockSpec((be, latent), lambda i: (i, 0))] * 3 \
        + [_full(w.shape) for w in pe]
    fn = pl.pallas_call(
        _edge_body,
        grid=(nsteps,),
        in_specs=in_specs,
        out_specs=pl.BlockSpec((be, latent), lambda i: (i, 0)),
        out_shape=jax.ShapeDtypeStruct((e, latent), jnp.float32),
        compiler_params=pltpu.CompilerParams(
            dimension_semantics=("parallel",)),
    )
    return fn(h_src, h_dst, edges, *pe)


def _sc_scatter(edges2, dst, n):
    e, latent = edges2.shape
    q = ((n + 3) // 4 + 127) // 128 * 128
    acc_rows = q + 128
    trash = q + 8
    zr = acc_rows // _NS
    cr = q // _NS
    epw = e // _NS
    nblk = epw // 128
    tail = epw - nblk * 128

    mesh = plsc.VectorSubcoreMesh(core_axis_name="c", subcore_axis_name="s",
                                  num_cores=_NC, num_subcores=_NS)

    scratch = [
        pltpu.VMEM_SHARED((acc_rows, latent), jnp.float32),
        pltpu.VMEM((zr, latent), jnp.float32),
        pltpu.VMEM((128,), jnp.int32),
        pltpu.VMEM((128,), jnp.int32),
        pltpu.VMEM((128, latent), jnp.float32),
    ]
    if tail:
        scratch.append(pltpu.VMEM((tail,), jnp.int32))

    @functools.partial(
        pl.kernel,
        out_type=jax.ShapeDtypeStruct((4 * q, latent), jnp.float32),
        mesh=mesh,
        scratch_types=scratch,
        compiler_params=pltpu.CompilerParams(use_tc_tiling_on_sc=False),
    )
    def k(e2, gr, em, acc, zbuf, dstb, idx1, vals, *maybe_tail):
        c = lax.axis_index("c")
        s = lax.axis_index("s")

        zv = jnp.zeros((16,), jnp.float32)

        def zrow(i, _):
            for qq in range(latent // 16):
                zbuf[i, pl.ds(qq * 16, 16)] = zv
            return 0

        lax.fori_loop(0, zr, zrow, 0)

        def make_idx(nrows, idx_ref, base_id):
            for kk in range(nrows // 16):
                d = dstb[pl.ds(kk * 16, 16)]
                li = d - base_id
                ok = (li >= 0) & (li < q)
                idx_ref[pl.ds(kk * 16, 16)] = jnp.where(ok, li, trash)

        for p in range(2):
            rid = c * 2 + p
            base_id = rid * q
            pltpu.sync_copy(zbuf, acc.at[pl.ds(s * zr, zr), :])
            plsc.subcore_barrier()

            def blk(b, _):
                off = s * epw + b * 128
                pltpu.sync_copy(gr.at[pl.ds(off, 128)], dstb)
                pltpu.sync_copy(e2.at[pl.ds(off, 128), :], vals)
                make_idx(128, idx1, base_id)
                pltpu.sync_copy(vals, acc.at[idx1], add=True)
                return 0

            lax.fori_loop(0, nblk, blk, 0)
            if tail:
                idxt = maybe_tail[0]
                off = s * epw + nblk * 128
                pltpu.sync_copy(gr.at[pl.ds(off, tail)],
                                dstb.at[pl.ds(0, tail)])
                pltpu.sync_copy(e2.at[pl.ds(off, tail), :],
                                vals.at[pl.ds(0, tail), :])
                make_idx(tail, idxt, base_id)
                pltpu.sync_copy(vals.at[pl.ds(0, tail), :],
                                acc.at[idxt], add=True)

            plsc.subcore_barrier()
            pltpu.sync_copy(acc.at[pl.ds(s * cr, cr), :],
                            em.at[pl.ds(base_id + s * cr, cr), :])
            plsc.subcore_barrier()

    return k(edges2, dst), q


def _stage_d_body(n1_ref, em_ref, W1, b1, g1, t1, W2, b2, g2, t2, out_ref):
    n1 = n1_ref[...]
    em = em_ref[...]
    cat = jnp.concatenate([n1, em], axis=-1)
    h = jnp.tanh(_ln(jnp.dot(cat, W1[...], preferred_element_type=jnp.float32)
                     + b1[...], g1[...], t1[...]))
    out = jnp.tanh(_ln(jnp.dot(h, W2[...], preferred_element_type=jnp.float32)
                       + b2[...], g2[...], t2[...])) + n1
    out_ref[...] = out


def _stage_d(nodes1, em, p_node2):
    n, latent = nodes1.shape
    bn = 2000 if n % 2000 == 0 else n
    nsteps = n // bn
    r2 = lambda a: a.reshape(1, -1)
    p2 = [p_node2[0], r2(p_node2[1]), r2(p_node2[2]), r2(p_node2[3]),
          p_node2[4], r2(p_node2[5]), r2(p_node2[6]), r2(p_node2[7])]
    in_specs = [pl.BlockSpec((bn, latent), lambda i: (i, 0))] * 2 \
        + [_full(w.shape) for w in p2]
    fn = pl.pallas_call(
        _stage_d_body,
        grid=(nsteps,),
        in_specs=in_specs,
        out_specs=pl.BlockSpec((bn, latent), lambda i: (i, 0)),
        out_shape=jax.ShapeDtypeStruct((n, latent), jnp.float32),
        compiler_params=pltpu.CompilerParams(
            dimension_semantics=("parallel",)),
    )
    return fn(nodes1, em, *p2)


def kernel(x, nodes, edges, assignment, supernodes, graph,
           p_node1, p_node2, p_edge, p_super, p_assign):
    n = nodes.shape[0]
    src = graph[0]
    dst = graph[1]
    nodes1, ua, a2, _nm, sn2 = _stage_a(
        assignment, nodes, x, supernodes, p_node1, p_assign, p_super)
    h_src, h_dst = _sc_gather(nodes1, src, dst)
    edges2 = _edge_mlp(h_src, h_dst, edges, p_edge)
    em_pad, _half = _sc_scatter(edges2, dst, n)
    nodes2 = _stage_d(nodes1, em_pad, p_node2)
    return (nodes2, edges2, a2, sn2, ua)


# speedup vs baseline: 1.1550x; 1.1550x over previous
import functools

import jax
import jax.numpy as jnp
from jax import lax
from jax.experimental import pallas as pl
from jax.experimental.pallas import tpu as pltpu
from jax.experimental.pallas import tpu_sc as plsc

_NC = 2
_NS = 16
_NW = _NC * _NS


def _ln(z, g, b):
    m = jnp.mean(z, axis=-1, keepdims=True)
    v = jnp.var(z, axis=-1, keepdims=True)
    return (z - m) * lax.rsqrt(v + 1e-5) * g + b


def _stage_a_body(nsteps,
                  a_ref, n_ref, x_ref, sn_ref,
                  W1, b1, g1, t1, W2, b2, g2, t2,
                  Wa1, ba1, ga1, ta1, Wa2, ba2,
                  Ws1, bs1, gs1, ts1, Ws2, bs2, gs2, ts2,
                  n1_ref, ua_ref, a2_ref, nm_ref, sn2_ref):
    i = pl.program_id(0)
    a = a_ref[...]
    sn = sn_ref[...]
    sm = jnp.dot(a, sn, preferred_element_type=jnp.float32)
    n0 = n_ref[...]
    cat = jnp.concatenate([n0, sm], axis=-1)
    h = jnp.tanh(_ln(jnp.dot(cat, W1[...], preferred_element_type=jnp.float32)
                     + b1[...], g1[...], t1[...]))
    n1 = jnp.tanh(_ln(jnp.dot(h, W2[...], preferred_element_type=jnp.float32)
                      + b2[...], g2[...], t2[...])) + n0
    n1_ref[...] = n1

    cat2 = jnp.concatenate([n1, x_ref[...]], axis=-1)
    ha = jnp.tanh(_ln(jnp.dot(cat2, Wa1[...], preferred_element_type=jnp.float32)
                      + ba1[...], ga1[...], ta1[...]))
    ua = jnp.dot(ha, Wa2[...], preferred_element_type=jnp.float32) + ba2[...]
    ua_ref[...] = ua
    mx = jnp.max(ua, axis=-1, keepdims=True)
    ex = jnp.exp(ua - mx)
    a2 = ex / jnp.sum(ex, axis=-1, keepdims=True)
    a2_ref[...] = a2

    part = lax.dot_general(a2, n1, (((0,), (0,)), ((), ())),
                           preferred_element_type=jnp.float32)

    @pl.when(i == 0)
    def _():
        nm_ref[...] = part

    @pl.when(i > 0)
    def _():
        nm_ref[...] += part

    @pl.when(i == nsteps - 1)
    def _():
        nm = nm_ref[...]
        cat3 = jnp.concatenate([sn, nm], axis=-1)
        hs = jnp.tanh(_ln(jnp.dot(cat3, Ws1[...], preferred_element_type=jnp.float32)
                          + bs1[...], gs1[...], ts1[...]))
        sn2 = jnp.tanh(_ln(jnp.dot(hs, Ws2[...], preferred_element_type=jnp.float32)
                           + bs2[...], gs2[...], ts2[...])) + sn
        sn2_ref[...] = sn2


def _full(shape):
    return pl.BlockSpec(shape, lambda i: (0, 0))


def _stage_a(assignment, nodes, x, supernodes, p_node1, p_assign, p_super):
    n, s = assignment.shape
    latent = nodes.shape[1]
    xdim = x.shape[1]
    bn = 1000 if n % 1000 == 0 else n
    nsteps = n // bn
    r2 = lambda a: a.reshape(1, -1)
    pn = [p_node1[0], r2(p_node1[1]), r2(p_node1[2]), r2(p_node1[3]),
          p_node1[4], r2(p_node1[5]), r2(p_node1[6]), r2(p_node1[7])]
    pa = [p_assign[0], r2(p_assign[1]), r2(p_assign[2]), r2(p_assign[3]),
          p_assign[4], r2(p_assign[5])]
    ps = [p_super[0], r2(p_super[1]), r2(p_super[2]), r2(p_super[3]),
          p_super[4], r2(p_super[5]), r2(p_super[6]), r2(p_super[7])]
    in_specs = [
        pl.BlockSpec((bn, s), lambda i: (i, 0)),
        pl.BlockSpec((bn, latent), lambda i: (i, 0)),
        pl.BlockSpec((bn, xdim), lambda i: (i, 0)),
        _full(supernodes.shape),
    ] + [_full(w.shape) for w in pn + pa + ps]
    out_specs = [
        pl.BlockSpec((bn, latent), lambda i: (i, 0)),
        pl.BlockSpec((bn, s), lambda i: (i, 0)),
        pl.BlockSpec((bn, s), lambda i: (i, 0)),
        _full((s, latent)),
        _full((s, latent)),
    ]
    out_shape = [
        jax.ShapeDtypeStruct((n, latent), jnp.float32),
        jax.ShapeDtypeStruct((n, s), jnp.float32),
        jax.ShapeDtypeStruct((n, s), jnp.float32),
        jax.ShapeDtypeStruct((s, latent), jnp.float32),
        jax.ShapeDtypeStruct((s, latent), jnp.float32),
    ]
    fn = pl.pallas_call(
        functools.partial(_stage_a_body, nsteps),
        grid=(nsteps,),
        in_specs=in_specs,
        out_specs=out_specs,
        out_shape=out_shape,
        compiler_params=pltpu.CompilerParams(
            dimension_semantics=("arbitrary",)),
    )
    return fn(assignment, nodes, x, supernodes, *pn, *pa, *ps)


def _sc_gather(nodes1, src, dst):
    e = src.shape[0]
    latent = nodes1.shape[1]
    epw = e // _NW
    gblk = 1280
    nblk = epw // gblk
    rem = epw - nblk * gblk
    nsub = gblk // 128
    t_total = 2 * nblk

    mesh = plsc.VectorSubcoreMesh(core_axis_name="c", subcore_axis_name="s",
                                  num_cores=_NC, num_subcores=_NS)

    @functools.partial(
        pl.kernel,
        out_type=(jax.ShapeDtypeStruct((e, latent), jnp.float32),
                  jax.ShapeDtypeStruct((e, latent), jnp.float32)),
        mesh=mesh,
        scratch_types=[
            pltpu.VMEM((2, gblk), jnp.int32),
            pltpu.VMEM((gblk, latent), jnp.float32),
            pltpu.VMEM((gblk, latent), jnp.float32),
            [pltpu.SemaphoreType.DMA] * 2,
            [pltpu.SemaphoreType.DMA] * 2,
            [pltpu.SemaphoreType.DMA] * 2,
        ],
        compiler_params=pltpu.CompilerParams(use_tc_tiling_on_sc=False),
    )
    def k(tbl, srcr, dstr, hs, hd, idxv, rows0, rows1, isems, gsems, osems):
        c = lax.axis_index("c")
        s = lax.axis_index("s")
        wid = s * _NC + c
        base = wid * epw
        rows = [rows0, rows1]

        def item_off(t):
            bt = jnp.where(t < nblk, t, t - nblk)
            return base + bt * gblk

        def fire_idx(t, par):
            off = item_off(t)

            @pl.when(t < nblk)
            def _():
                pltpu.async_copy(srcr.at[pl.ds(off, gblk)],
                                 idxv.at[par], isems[par])

            @pl.when(t >= nblk)
            def _():
                pltpu.async_copy(dstr.at[pl.ds(off, gblk)],
                                 idxv.at[par], isems[par])

        def wait_idx(par):
            pltpu.make_async_copy(srcr.at[pl.ds(0, gblk)],
                                  idxv.at[par], isems[par]).wait()

        def fire_gathers(t, par):
            del t
            for j in range(nsub):
                pltpu.async_copy(
                    tbl.at[idxv.at[par, pl.ds(j * 128, 128)]],
                    rows[par].at[pl.ds(j * 128, 128), :], gsems[par])

        def wait_gathers(par):
            pltpu.make_async_copy(tbl.at[pl.ds(0, gblk)],
                                  rows[par], gsems[par]).wait()

        def fire_out(t, par):
            off = item_off(t)

            @pl.when(t < nblk)
            def _():
                pltpu.async_copy(rows[par], hs.at[pl.ds(off, gblk), :],
                                 osems[par])

            @pl.when(t >= nblk)
            def _():
                pltpu.async_copy(rows[par], hd.at[pl.ds(off, gblk), :],
                                 osems[par])

        def wait_out(par):
            pltpu.make_async_copy(tbl.at[pl.ds(0, gblk)],
                                  rows[par], osems[par]).wait()

        fire_idx(jnp.int32(0), 0)
        if t_total > 1:
            fire_idx(jnp.int32(1), 1)

        def step(t, _):
            par = lax.rem(t, 2)

            @pl.when(par == 0)
            def _():
                _step(t, 0)

            @pl.when(par == 1)
            def _():
                _step(t, 1)

            return 0

        def _step(t, par):
            wait_idx(par)

            @pl.when(t >= 2)
            def _():
                wait_out(par)

            fire_gathers(t, par)

            @pl.when(t >= 1)
            def _():
                wait_gathers(1 - par)

                @pl.when(t + 1 < t_total)
                def _():
                    fire_idx(t + 1, 1 - par)

                fire_out(t - 1, 1 - par)

        lax.fori_loop(0, t_total, step, 0)

        last = t_total - 1
        lpar = last % 2
        wait_gathers(lpar)
        fire_out(jnp.int32(last), lpar)
        wait_out(1 - lpar)
        wait_out(lpar)

        if rem:
            def do_rem(gr, out):
                off = base + nblk * gblk
                pltpu.sync_copy(gr.at[pl.ds(off, rem)],
                                idxv.at[0, pl.ds(0, rem)])
                descs = []
                p = 0
                while p < rem:
                    kk = min(128, rem - p)
                    descs.append(pltpu.async_copy(
                        tbl.at[idxv.at[0, pl.ds(p, kk)]],
                        rows0.at[pl.ds(p, kk), :], gsems[0]))
                    p += kk
                for d in descs:
                    d.wait()
                pltpu.sync_copy(rows0.at[pl.ds(0, rem), :],
                                out.at[pl.ds(off, rem), :])

            do_rem(srcr, hs)
            do_rem(dstr, hd)

    return k(nodes1, src, dst)


def _edge_body(latent, hs_ref, hd_ref, e_ref,
               W1, b1, g1, t1, W2, b2, g2, t2, out_ref):
    hs4 = hs_ref[...]
    hd4 = hd_ref[...]
    e4 = e_ref[...]
    W1f = W1[...]
    W2f = W2[...]
    outs = []
    for i in range(4):
        sl = slice(i * latent, (i + 1) * latent)
        z = (jnp.dot(hs4[:, sl], W1f[0:latent, :],
                     preferred_element_type=jnp.float32)
             + jnp.dot(hd4[:, sl], W1f[latent:2 * latent, :],
                       preferred_element_type=jnp.float32)
             + jnp.dot(e4[:, sl], W1f[2 * latent:3 * latent, :],
                       preferred_element_type=jnp.float32)
             + b1[...])
        h = jnp.tanh(_ln(z, g1[...], t1[...]))
        o = jnp.tanh(_ln(jnp.dot(h, W2f, preferred_element_type=jnp.float32)
                         + b2[...], g2[...], t2[...]))
        outs.append(o)
    out_ref[...] = jnp.concatenate(outs, axis=-1) + e4


def _edge_mlp(h_src4, h_dst4, edges4, p_edge):
    ep, lanes = edges4.shape
    latent = lanes // 4
    bp = 1600 if ep % 1600 == 0 else ep
    nsteps = ep // bp
    r2 = lambda a: a.reshape(1, -1)
    pe = [p_edge[0], r2(p_edge[1]), r2(p_edge[2]), r2(p_edge[3]),
          p_edge[4], r2(p_edge[5]), r2(p_edge[6]), r2(p_edge[7])]
    in_specs = [pl.BlockSpec((bp, lanes), lambda i: (i, 0))] * 3 \
        + [_full(w.shape) for w in pe]
    fn = pl.pallas_call(
        functools.partial(_edge_body, latent),
        grid=(nsteps,),
        in_specs=in_specs,
        out_specs=pl.BlockSpec((bp, lanes), lambda i: (i, 0)),
        out_shape=jax.ShapeDtypeStruct((ep, lanes), jnp.float32),
        compiler_params=pltpu.CompilerParams(
            dimension_semantics=("parallel",)),
    )
    return fn(h_src4, h_dst4, edges4, *pe)


def _sc_scatter(edges2, dst, n):
    e, latent = edges2.shape
    q = ((n + 3) // 4 + 127) // 128 * 128
    acc_rows = q + 128
    trash = q + 8
    zr = acc_rows // _NS
    cr = q // _NS
    epw = e // _NS
    sblk = 384
    nsub = sblk // 128
    nblk = epw // sblk
    tail = epw - nblk * sblk
    tail_rem = tail % 128

    mesh = plsc.VectorSubcoreMesh(core_axis_name="c", subcore_axis_name="s",
                                  num_cores=_NC, num_subcores=_NS)

    scratch = [
        pltpu.VMEM_SHARED((acc_rows, latent), jnp.float32),
        pltpu.VMEM((zr, latent), jnp.float32),
        pltpu.VMEM((2, sblk), jnp.int32),
        pltpu.VMEM((sblk, latent), jnp.float32),
        pltpu.VMEM((sblk, latent), jnp.float32),
        pltpu.VMEM((nsub, 128), jnp.int32),
        pltpu.VMEM((nsub, 128), jnp.int32),
        [pltpu.SemaphoreType.DMA] * 2,
        [pltpu.SemaphoreType.DMA] * 2,
        pltpu.VMEM((128,), jnp.int32),
        pltpu.VMEM((max(tail_rem, 8),), jnp.int32),
    ]

    @functools.partial(
        pl.kernel,
        out_type=jax.ShapeDtypeStruct((4 * q, latent), jnp.float32),
        mesh=mesh,
        scratch_types=scratch,
        compiler_params=pltpu.CompilerParams(use_tc_tiling_on_sc=False),
    )
    def k(e2, gr, em, acc, zbuf, dstb, vals0, vals1, idx0, idx1,
          lsems, asems, idxt, idxr):
        c = lax.axis_index("c")
        s = lax.axis_index("s")
        vals = [vals0, vals1]
        idx2 = [idx0, idx1]

        zv = jnp.zeros((16,), jnp.float32)

        def zrow(i, _):
            for qq in range(latent // 16):
                zbuf[i, pl.ds(qq * 16, 16)] = zv
            return 0

        lax.fori_loop(0, zr, zrow, 0)

        def fire_loads(b, par):
            off = s * epw + b * sblk
            pltpu.async_copy(gr.at[pl.ds(off, sblk)], dstb.at[par],
                             lsems[par])
            pltpu.async_copy(e2.at[pl.ds(off, sblk), :], vals[par],
                             lsems[par])

        def wait_loads(par):
            pltpu.make_async_copy(gr.at[pl.ds(0, sblk)], dstb.at[par],
                                  lsems[par]).wait()
            pltpu.make_async_copy(e2.at[pl.ds(0, sblk), :], vals[par],
                                  lsems[par]).wait()

        def clamp16(d, base_id):
            li = d - base_id
            ok = (li >= 0) & (li < q)
            return jnp.where(ok, li, trash)

        def make_idx(par, base_id):
            for j in range(nsub):
                for kk in range(8):
                    d = dstb[par, pl.ds(j * 128 + kk * 16, 16)]
                    idx2[par][j, pl.ds(kk * 16, 16)] = clamp16(d, base_id)

        def fire_adds(par):
            for j in range(nsub):
                pltpu.async_copy(vals[par].at[pl.ds(j * 128, 128), :],
                                 acc.at[idx2[par].at[j]], asems[par],
                                 add=True)

        def wait_adds(par):
            pltpu.make_async_copy(e2.at[pl.ds(0, sblk), :], vals[par],
                                  asems[par]).wait()

        for p in range(2):
            rid = c * 2 + p
            base_id = rid * q
            pltpu.sync_copy(zbuf, acc.at[pl.ds(s * zr, zr), :])
            plsc.subcore_barrier()

            fire_loads(jnp.int32(0), 0)

            def step(b, _):
                @pl.when(lax.rem(b, 2) == 0)
                def _():
                    body(b, 0)

                @pl.when(lax.rem(b, 2) == 1)
                def _():
                    body(b, 1)

                return 0

            def body(b, par):
                wait_loads(par)
                make_idx(par, base_id)
                fire_adds(par)

                @pl.when(b + 1 < nblk)
                def _():
                    @pl.when(b >= 1)
                    def _():
                        wait_adds(1 - par)

                    fire_loads(b + 1, 1 - par)

            lax.fori_loop(0, nblk, step, 0)
            if nblk >= 2:
                wait_adds(nblk % 2)
            if nblk >= 1:
                wait_adds(1 - nblk % 2)

            tp = 0
            while tp < tail:
                kk = min(128, tail - tp)
                off = s * epw + nblk * sblk + tp
                pltpu.sync_copy(gr.at[pl.ds(off, kk)],
                                dstb.at[0, pl.ds(0, kk)])
                pltpu.sync_copy(e2.at[pl.ds(off, kk), :],
                                vals0.at[pl.ds(0, kk), :])
                iref = idxt if kk == 128 else idxr
                for t16 in range(kk // 16):
                    d = dstb[0, pl.ds(t16 * 16, 16)]
                    iref[pl.ds(t16 * 16, 16)] = clamp16(d, base_id)
                pltpu.sync_copy(vals0.at[pl.ds(0, kk), :],
                                acc.at[iref], add=True)
                tp += kk

            plsc.subcore_barrier()
            pltpu.sync_copy(acc.at[pl.ds(s * cr, cr), :],
                            em.at[pl.ds(base_id + s * cr, cr), :])
            plsc.subcore_barrier()

    return k(edges2, dst), q


def _stage_d_body(n1_ref, em_ref, W1, b1, g1, t1, W2, b2, g2, t2, out_ref):
    n1 = n1_ref[...]
    em = em_ref[...]
    cat = jnp.concatenate([n1, em], axis=-1)
    h = jnp.tanh(_ln(jnp.dot(cat, W1[...], preferred_element_type=jnp.float32)
                     + b1[...], g1[...], t1[...]))
    out = jnp.tanh(_ln(jnp.dot(h, W2[...], preferred_element_type=jnp.float32)
                       + b2[...], g2[...], t2[...])) + n1
    out_ref[...] = out


def _stage_d(nodes1, em, p_node2):
    n, latent = nodes1.shape
    bn = 2000 if n % 2000 == 0 else n
    nsteps = n // bn
    r2 = lambda a: a.reshape(1, -1)
    p2 = [p_node2[0], r2(p_node2[1]), r2(p_node2[2]), r2(p_node2[3]),
          p_node2[4], r2(p_node2[5]), r2(p_node2[6]), r2(p_node2[7])]
    in_specs = [pl.BlockSpec((bn, latent), lambda i: (i, 0))] * 2 \
        + [_full(w.shape) for w in p2]
    fn = pl.pallas_call(
        _stage_d_body,
        grid=(nsteps,),
        in_specs=in_specs,
        out_specs=pl.BlockSpec((bn, latent), lambda i: (i, 0)),
        out_shape=jax.ShapeDtypeStruct((n, latent), jnp.float32),
        compiler_params=pltpu.CompilerParams(
            dimension_semantics=("parallel",)),
    )
    return fn(nodes1, em, *p2)


def kernel(x, nodes, edges, assignment, supernodes, graph,
           p_node1, p_node2, p_edge, p_super, p_assign):
    n = nodes.shape[0]
    src = graph[0]
    dst = graph[1]
    e, latent = edges.shape
    nodes1, ua, a2, _nm, sn2 = _stage_a(
        assignment, nodes, x, supernodes, p_node1, p_assign, p_super)
    h_src, h_dst = _sc_gather(nodes1, src, dst)
    edges2_4 = _edge_mlp(h_src.reshape(e // 4, 128),
                         h_dst.reshape(e // 4, 128),
                         edges.reshape(e // 4, 128), p_edge)
    edges2 = edges2_4.reshape(e, latent)
    em_pad, _half = _sc_scatter(edges2, dst, n)
    nodes2 = _stage_d(nodes1, em_pad, p_node2)
    return (nodes2, edges2, a2, sn2, ua)
